# baseline (speedup 1.0000x reference)
import functools

import jax
import jax.numpy as jnp
from jax.experimental import pallas as pl
from jax.experimental.pallas import tpu as pltpu

_BR = 512


def _airtnn_body(mat_ref, x_ref, wt_ref, wh_ref, out_ref,
                 cache_ref, ybuf_ref, *, nblk, taps):
    m = pl.program_id(0)
    t = pl.program_id(1)
    i = pl.program_id(2)
    rows = pl.ds(i * _BR, _BR)

    @pl.when((t == 0) & (i == 0))
    def _():
        ybuf_ref[0] = x_ref[:].astype(jnp.bfloat16)

    rd = t % 2

    @pl.when(t == 0)
    def _():
        strip = mat_ref[0]
        cache_ref[i] = strip
        yf = jax.lax.dot(strip, ybuf_ref[rd],
                         preferred_element_type=jnp.float32)
        ybuf_ref[1 - rd, rows, :] = yf.astype(jnp.bfloat16)
        contrib = jax.lax.dot(yf, wt_ref[0, 0],
                              preferred_element_type=jnp.float32)

        @pl.when(m == 0)
        def _():
            out_ref[rows, :] = contrib + jax.lax.dot(
                x_ref[rows, :], wh_ref[:],
                preferred_element_type=jnp.float32)

        @pl.when(m != 0)
        def _():
            out_ref[rows, :] = out_ref[rows, :] + contrib

    @pl.when(t != 0)
    def _():
        yf = jax.lax.dot(cache_ref[i], ybuf_ref[rd],
                         preferred_element_type=jnp.float32)
        if taps > 1:
            ybuf_ref[1 - rd, rows, :] = yf.astype(jnp.bfloat16)
        out_ref[rows, :] = out_ref[rows, :] + jax.lax.dot(
            yf, wt_ref[0, 0], preferred_element_type=jnp.float32)


def kernel(x, lower_lp, upper_lp, W_up, W_low, W_h):
    B, N, C_in = x.shape
    T, C_out, _ = W_up.shape
    BC = B * C_in
    BCO = B * C_out
    nblk = N // _BR

    mats = jnp.stack([upper_lp, lower_lp]).astype(jnp.bfloat16)
    xw = jnp.transpose(x, (1, 0, 2)).reshape(N, BC)

    eye = jnp.eye(B, dtype=jnp.float32)
    wt = jnp.stack([
        jnp.stack([jnp.kron(eye, W_up[t].T) for t in range(T)]),
        jnp.stack([jnp.kron(eye, W_low[t].T) for t in range(T)]),
    ])
    wh = jnp.kron(eye, W_h.T)

    out = pl.pallas_call(
        functools.partial(_airtnn_body, nblk=nblk, taps=T),
        grid=(2, T, nblk),
        in_specs=[
            pl.BlockSpec((1, _BR, N),
                         lambda m, t, i: (m, jnp.where(t == 0, i, nblk - 1), 0)),
            pl.BlockSpec((N, BC), lambda m, t, i: (0, 0)),
            pl.BlockSpec((1, 1, BC, BCO), lambda m, t, i: (m, t, 0, 0)),
            pl.BlockSpec((BC, BCO), lambda m, t, i: (0, 0)),
        ],
        out_specs=pl.BlockSpec((N, BCO), lambda m, t, i: (0, 0)),
        out_shape=jax.ShapeDtypeStruct((N, BCO), jnp.float32),
        scratch_shapes=[
            pltpu.VMEM((nblk, _BR, N), jnp.bfloat16),
            pltpu.VMEM((2, N, BC), jnp.bfloat16),
        ],
        compiler_params=pltpu.CompilerParams(
            dimension_semantics=("arbitrary", "arbitrary", "arbitrary"),
            vmem_limit_bytes=60 * 1024 * 1024,
        ),
    )(mats, xw, wt, wh)

    return jnp.transpose(out.reshape(N, B, C_out), (1, 0, 2))


# speedup vs baseline: 1.0157x; 1.0157x over previous
import functools

import jax
import jax.numpy as jnp
from jax.experimental import pallas as pl
from jax.experimental.pallas import tpu as pltpu

_BR = 512


def _airtnn_body(mats_ref, x_ref, wt_ref, wh_ref, out_ref,
                 cache_ref, ybuf_ref, sem_ref, *, nblk, taps):
    def copy(m, i):
        return pltpu.make_async_copy(
            mats_ref.at[m, pl.ds(i * _BR, _BR), :],
            cache_ref.at[i],
            sem_ref.at[m, i],
        )

    def start0(i, _):
        copy(0, i).start()
        return _

    jax.lax.fori_loop(0, nblk, start0, 0)

    for m in range(2):
        ybuf_ref[0] = x_ref[:].astype(jnp.bfloat16)
        for t in range(taps):
            rd, wr = t % 2, (t + 1) % 2
            first = m == 0 and t == 0
            last_of_m0 = m == 0 and t == taps - 1
            very_last = m == 1 and t == taps - 1

            def strip(i, carry, m=m, t=t, rd=rd, wr=wr, first=first,
                      last_of_m0=last_of_m0, very_last=very_last):
                rows = pl.ds(i * _BR, _BR)
                if t == 0:
                    copy(m, i).wait()
                yf = jax.lax.dot(cache_ref[i], ybuf_ref[rd],
                                 preferred_element_type=jnp.float32)
                if not very_last:
                    ybuf_ref[wr, rows, :] = yf.astype(jnp.bfloat16)
                contrib = jax.lax.dot(yf, wt_ref[m, t],
                                      preferred_element_type=jnp.float32)
                if first:
                    out_ref[rows, :] = contrib + jax.lax.dot(
                        x_ref[rows, :], wh_ref[:],
                        preferred_element_type=jnp.float32)
                else:
                    out_ref[rows, :] = out_ref[rows, :] + contrib
                if last_of_m0:
                    @pl.when(i > 0)
                    def _prefetch():
                        copy(1, i - 1).start()
                return carry

            jax.lax.fori_loop(0, nblk, strip, 0)
            if last_of_m0:
                copy(1, nblk - 1).start()


def kernel(x, lower_lp, upper_lp, W_up, W_low, W_h):
    B, N, C_in = x.shape
    T, C_out, _ = W_up.shape
    BC = B * C_in
    BCO = B * C_out
    nblk = N // _BR

    mats = jnp.stack([upper_lp, lower_lp]).astype(jnp.bfloat16)
    xw = jnp.transpose(x, (1, 0, 2)).reshape(N, BC)

    eye = jnp.eye(B, dtype=jnp.float32)
    wt = jnp.stack([
        jnp.stack([jnp.kron(eye, W_up[t].T) for t in range(T)]),
        jnp.stack([jnp.kron(eye, W_low[t].T) for t in range(T)]),
    ])
    wh = jnp.kron(eye, W_h.T)

    out = pl.pallas_call(
        functools.partial(_airtnn_body, nblk=nblk, taps=T),
        in_specs=[
            pl.BlockSpec(memory_space=pl.ANY),
            pl.BlockSpec((N, BC), lambda: (0, 0)),
            pl.BlockSpec((2, T, BC, BCO), lambda: (0, 0, 0, 0)),
            pl.BlockSpec((BC, BCO), lambda: (0, 0)),
        ],
        out_specs=pl.BlockSpec((N, BCO), lambda: (0, 0)),
        out_shape=jax.ShapeDtypeStruct((N, BCO), jnp.float32),
        scratch_shapes=[
            pltpu.VMEM((nblk, _BR, N), jnp.bfloat16),
            pltpu.VMEM((2, N, BC), jnp.bfloat16),
            pltpu.SemaphoreType.DMA((2, nblk)),
        ],
        compiler_params=pltpu.CompilerParams(
            vmem_limit_bytes=60 * 1024 * 1024,
        ),
    )(mats, xw, wt, wh)

    return jnp.transpose(out.reshape(N, B, C_out), (1, 0, 2))


# speedup vs baseline: 1.6949x; 1.6687x over previous
import functools

import jax
import jax.numpy as jnp
from jax.experimental import pallas as pl
from jax.experimental.pallas import tpu as pltpu

_BR = 256
_NS = 4


def _airtnn_body(u_ref, l_ref, x_ref, wt_ref, wh_ref, out_ref,
                 stage_ref, cache_ref, ybuf_ref, sem_ref, *, nblk, taps):
    mrefs = (u_ref, l_ref)

    def copy(m, i):
        return pltpu.make_async_copy(
            mrefs[m].at[pl.ds(i * _BR, _BR), :],
            stage_ref.at[jax.lax.rem(i, _NS)],
            sem_ref.at[m, i],
        )

    for i in range(_NS):
        copy(0, i).start()

    for m in range(2):
        ybuf_ref[0] = x_ref[:].astype(jnp.bfloat16)
        for t in range(taps):
            rd, wr = t % 2, (t + 1) % 2
            first = m == 0 and t == 0
            very_last = m == 1 and t == taps - 1

            def strip(i, carry, m=m, t=t, rd=rd, wr=wr, first=first,
                      very_last=very_last):
                rows = pl.ds(i * _BR, _BR)
                if t == 0:
                    copy(m, i).wait()
                    cache_ref[i] = stage_ref[jax.lax.rem(i, _NS)].astype(
                        jnp.bfloat16)
                    if m == 0:
                        @pl.when(i + _NS < nblk)
                        def _next_same():
                            copy(0, i + _NS).start()

                        @pl.when(i + _NS >= nblk)
                        def _next_other():
                            copy(1, i + _NS - nblk).start()
                    else:
                        @pl.when(i + _NS < nblk)
                        def _next():
                            copy(1, i + _NS).start()
                yf = jax.lax.dot(cache_ref[i], ybuf_ref[rd],
                                 preferred_element_type=jnp.float32)
                if not very_last:
                    ybuf_ref[wr, rows, :] = yf.astype(jnp.bfloat16)
                contrib = jax.lax.dot(yf, wt_ref[m, t],
                                      preferred_element_type=jnp.float32)
                if first:
                    out_ref[rows, :] = contrib + jax.lax.dot(
                        x_ref[rows, :], wh_ref[:],
                        preferred_element_type=jnp.float32)
                else:
                    out_ref[rows, :] = out_ref[rows, :] + contrib
                return carry

            jax.lax.fori_loop(0, nblk, strip, 0)


def kernel(x, lower_lp, upper_lp, W_up, W_low, W_h):
    B, N, C_in = x.shape
    T, C_out, _ = W_up.shape
    BC = B * C_in
    BCO = B * C_out
    nblk = N // _BR

    xw = jnp.transpose(x, (1, 0, 2)).reshape(N, BC)

    eye = jnp.eye(B, dtype=jnp.float32)
    wt = jnp.stack([
        jnp.stack([jnp.kron(eye, W_up[t].T) for t in range(T)]),
        jnp.stack([jnp.kron(eye, W_low[t].T) for t in range(T)]),
    ])
    wh = jnp.kron(eye, W_h.T)

    out = pl.pallas_call(
        functools.partial(_airtnn_body, nblk=nblk, taps=T),
        in_specs=[
            pl.BlockSpec(memory_space=pl.ANY),
            pl.BlockSpec(memory_space=pl.ANY),
            pl.BlockSpec((N, BC), lambda: (0, 0)),
            pl.BlockSpec((2, T, BC, BCO), lambda: (0, 0, 0, 0)),
            pl.BlockSpec((BC, BCO), lambda: (0, 0)),
        ],
        out_specs=pl.BlockSpec((N, BCO), lambda: (0, 0)),
        out_shape=jax.ShapeDtypeStruct((N, BCO), jnp.float32),
        scratch_shapes=[
            pltpu.VMEM((_NS, _BR, N), jnp.float32),
            pltpu.VMEM((nblk, _BR, N), jnp.bfloat16),
            pltpu.VMEM((2, N, BC), jnp.bfloat16),
            pltpu.SemaphoreType.DMA((2, nblk)),
        ],
        compiler_params=pltpu.CompilerParams(
            vmem_limit_bytes=60 * 1024 * 1024,
        ),
    )(upper_lp, lower_lp, xw, wt, wh)

    return jnp.transpose(out.reshape(N, B, C_out), (1, 0, 2))


# speedup vs baseline: 1.7696x; 1.0441x over previous
import functools

import jax
import jax.numpy as jnp
from jax.experimental import pallas as pl
from jax.experimental.pallas import tpu as pltpu

_BR = 256
_NS = 4

_NT = (((1,), (1,)), ((), ()))


def _airtnn_body(u_ref, l_ref, xt_ref, wt_ref, wh_ref, out_ref,
                 stage_ref, cache_ref, ybuf_ref, sem_ref, *, nblk, taps):
    mrefs = (u_ref, l_ref)

    def copy(m, i):
        return pltpu.make_async_copy(
            mrefs[m].at[pl.ds(i * _BR, _BR), :],
            stage_ref.at[jax.lax.rem(i, _NS)],
            sem_ref.at[m, i],
        )

    for i in range(_NS):
        copy(0, i).start()

    for m in range(2):
        ybuf_ref[0] = xt_ref[:].astype(jnp.bfloat16)
        for t in range(taps):
            rd, wr = t % 2, (t + 1) % 2
            first = m == 0 and t == 0
            very_last = m == 1 and t == taps - 1

            def strip(i, carry, m=m, t=t, rd=rd, wr=wr, first=first,
                      very_last=very_last):
                cols = pl.ds(i * _BR, _BR)
                if t == 0:
                    copy(m, i).wait()
                    cache_ref[i] = stage_ref[jax.lax.rem(i, _NS)].astype(
                        jnp.bfloat16)
                    if m == 0:
                        @pl.when(i + _NS < nblk)
                        def _next_same():
                            copy(0, i + _NS).start()

                        @pl.when(i + _NS >= nblk)
                        def _next_other():
                            copy(1, i + _NS - nblk).start()
                    else:
                        @pl.when(i + _NS < nblk)
                        def _next():
                            copy(1, i + _NS).start()
                yf = jax.lax.dot_general(ybuf_ref[rd], cache_ref[i], _NT,
                                         preferred_element_type=jnp.float32)
                if not very_last:
                    ybuf_ref[wr, :, cols] = yf.astype(jnp.bfloat16)
                contrib = jax.lax.dot(wt_ref[m, t], yf,
                                      preferred_element_type=jnp.float32)
                if first:
                    out_ref[:, cols] = contrib + jax.lax.dot(
                        wh_ref[:], xt_ref[:, cols],
                        preferred_element_type=jnp.float32)
                else:
                    out_ref[:, cols] = out_ref[:, cols] + contrib
                return carry

            jax.lax.fori_loop(0, nblk, strip, 0)


def kernel(x, lower_lp, upper_lp, W_up, W_low, W_h):
    B, N, C_in = x.shape
    T, C_out, _ = W_up.shape
    BC = B * C_in
    BCO = B * C_out
    nblk = N // _BR

    xt = jnp.transpose(x, (0, 2, 1)).reshape(BC, N)

    eye = jnp.eye(B, dtype=jnp.float32)
    wt = jnp.stack([
        jnp.stack([jnp.kron(eye, W_up[t]) for t in range(T)]),
        jnp.stack([jnp.kron(eye, W_low[t]) for t in range(T)]),
    ])
    wh = jnp.kron(eye, W_h)

    out = pl.pallas_call(
        functools.partial(_airtnn_body, nblk=nblk, taps=T),
        in_specs=[
            pl.BlockSpec(memory_space=pl.ANY),
            pl.BlockSpec(memory_space=pl.ANY),
            pl.BlockSpec((BC, N), lambda: (0, 0)),
            pl.BlockSpec((2, T, BCO, BC), lambda: (0, 0, 0, 0)),
            pl.BlockSpec((BCO, BC), lambda: (0, 0)),
        ],
        out_specs=pl.BlockSpec((BCO, N), lambda: (0, 0)),
        out_shape=jax.ShapeDtypeStruct((BCO, N), jnp.float32),
        scratch_shapes=[
            pltpu.VMEM((_NS, _BR, N), jnp.float32),
            pltpu.VMEM((nblk, _BR, N), jnp.bfloat16),
            pltpu.VMEM((2, BC, N), jnp.bfloat16),
            pltpu.SemaphoreType.DMA((2, nblk)),
        ],
        compiler_params=pltpu.CompilerParams(
            vmem_limit_bytes=60 * 1024 * 1024,
        ),
    )(upper_lp, lower_lp, xt, wt, wh)

    return jnp.transpose(out.reshape(B, C_out, N), (0, 2, 1))


# speedup vs baseline: 1.9641x; 1.1099x over previous
import functools

import jax
import jax.numpy as jnp
from jax.experimental import pallas as pl
from jax.experimental.pallas import tpu as pltpu

_BR = 512
_NS = 2

_NT = (((1,), (1,)), ((), ()))


def _airtnn_body(u_ref, l_ref, xt_ref, wt_ref, wh_ref, out_ref,
                 stage_ref, cache_ref, ys_ref, ya_ref, yb_ref, sem_ref,
                 *, nblk, taps):
    mrefs = (u_ref, l_ref)

    def copy(m, i):
        return pltpu.make_async_copy(
            mrefs[m].at[pl.ds(i * _BR, _BR), :],
            stage_ref.at[jax.lax.rem(i, _NS)],
            sem_ref.at[m, i],
        )

    def land(m, i):
        copy(m, i).wait()
        cache_ref[i] = stage_ref[jax.lax.rem(i, _NS)].astype(jnp.bfloat16)
        if m == 0:
            @pl.when(i + _NS < nblk)
            def _next_same():
                copy(0, i + _NS).start()

            @pl.when(i + _NS >= nblk)
            def _next_other():
                copy(1, i + _NS - nblk).start()
        else:
            @pl.when(i + _NS < nblk)
            def _next():
                copy(1, i + _NS).start()

    for i in range(_NS):
        copy(0, i).start()

    ys_ref[:] = xt_ref[:].astype(jnp.bfloat16)

    bufs = (ys_ref, ya_ref, yb_ref, ya_ref)

    for m in range(2):
        for t in range(taps):
            rdb, wrb = bufs[t], bufs[t + 1]
            first = m == 0 and t == 0
            last_t = t == taps - 1

            def strip(i, carry, m=m, t=t, rdb=rdb, wrb=wrb, first=first,
                      last_t=last_t):
                cols = pl.ds(i * _BR, _BR)
                if m == 0 and t == 0:
                    land(0, i)
                yf = jax.lax.dot_general(rdb[:], cache_ref[i], _NT,
                                         preferred_element_type=jnp.float32)
                if m == 0 and last_t:
                    land(1, i)
                if not last_t:
                    wrb[:, cols] = yf.astype(jnp.bfloat16)
                contrib = jax.lax.dot(wt_ref[m, t], yf,
                                      preferred_element_type=jnp.float32)
                if first:
                    out_ref[:, cols] = contrib + jax.lax.dot(
                        wh_ref[:], xt_ref[:, cols],
                        preferred_element_type=jnp.float32)
                else:
                    out_ref[:, cols] = out_ref[:, cols] + contrib
                return carry

            jax.lax.fori_loop(0, nblk, strip, 0)


def kernel(x, lower_lp, upper_lp, W_up, W_low, W_h):
    B, N, C_in = x.shape
    T, C_out, _ = W_up.shape
    BC = B * C_in
    BCO = B * C_out
    nblk = N // _BR

    xt = jnp.transpose(x, (0, 2, 1)).reshape(BC, N)

    eye = jnp.eye(B, dtype=jnp.float32)
    wt = jnp.stack([
        jnp.stack([jnp.kron(eye, W_up[t]) for t in range(T)]),
        jnp.stack([jnp.kron(eye, W_low[t]) for t in range(T)]),
    ])
    wh = jnp.kron(eye, W_h)

    out = pl.pallas_call(
        functools.partial(_airtnn_body, nblk=nblk, taps=T),
        in_specs=[
            pl.BlockSpec(memory_space=pl.ANY),
            pl.BlockSpec(memory_space=pl.ANY),
            pl.BlockSpec((BC, N), lambda: (0, 0)),
            pl.BlockSpec((2, T, BCO, BC), lambda: (0, 0, 0, 0)),
            pl.BlockSpec((BCO, BC), lambda: (0, 0)),
        ],
        out_specs=pl.BlockSpec((BCO, N), lambda: (0, 0)),
        out_shape=jax.ShapeDtypeStruct((BCO, N), jnp.float32),
        scratch_shapes=[
            pltpu.VMEM((_NS, _BR, N), jnp.float32),
            pltpu.VMEM((nblk, _BR, N), jnp.bfloat16),
            pltpu.VMEM((BC, N), jnp.bfloat16),
            pltpu.VMEM((BC, N), jnp.bfloat16),
            pltpu.VMEM((BC, N), jnp.bfloat16),
            pltpu.SemaphoreType.DMA((2, nblk)),
        ],
        compiler_params=pltpu.CompilerParams(
            vmem_limit_bytes=60 * 1024 * 1024,
        ),
    )(upper_lp, lower_lp, xt, wt, wh)

    return jnp.transpose(out.reshape(B, C_out, N), (0, 2, 1))


# speedup vs baseline: 2.2361x; 1.1385x over previous
import functools

import jax
import jax.numpy as jnp
from jax.experimental import pallas as pl
from jax.experimental.pallas import tpu as pltpu

_BR = 512
_NS = 2

_NT = (((1,), (1,)), ((), ()))


def _airtnn_body(u_ref, l_ref, xt_ref, wt_ref, wh_ref, out_ref,
                 stage_ref, cache_ref, ys_ref, ya_ref, yb_ref, sem_ref,
                 *, nblk, taps):
    mrefs = (u_ref, l_ref)

    def copy(m, i):
        return pltpu.make_async_copy(
            mrefs[m].at[pl.ds(i * _BR, _BR), :],
            stage_ref.at[i % _NS],
            sem_ref.at[m, i],
        )

    def land(m, i):
        copy(m, i).wait()
        cache_ref[i] = stage_ref[i % _NS].astype(jnp.bfloat16)
        if m == 0:
            if i + _NS < nblk:
                copy(0, i + _NS).start()
            else:
                copy(1, i + _NS - nblk).start()
        elif i + _NS < nblk:
            copy(1, i + _NS).start()

    def tap(m, t, rdb, i, write_to=None):
        cols = pl.ds(i * _BR, _BR)
        yf = jax.lax.dot_general(rdb[:], cache_ref[i], _NT,
                                 preferred_element_type=jnp.float32)
        if write_to is not None:
            write_to[:, cols] = yf.astype(jnp.bfloat16)
        contrib = jax.lax.dot(wt_ref[m, t], yf,
                              preferred_element_type=jnp.float32)
        if m == 0 and t == 0:
            out_ref[:, cols] = contrib + jax.lax.dot(
                wh_ref[:], xt_ref[:, cols],
                preferred_element_type=jnp.float32)
        else:
            out_ref[:, cols] = out_ref[:, cols] + contrib

    for i in range(_NS):
        copy(0, i).start()

    ys_ref[:] = xt_ref[:].astype(jnp.bfloat16)

    for i in range(nblk):
        land(0, i)
        tap(0, 0, ys_ref, i, write_to=ya_ref)

    rdb, wrb = ya_ref, yb_ref
    for t in range(1, taps - 1):
        for i in range(nblk):
            tap(0, t, rdb, i, write_to=wrb)
        rdb, wrb = wrb, rdb

    for i in range(nblk):
        tap(0, taps - 1, rdb, i)
        land(1, i)
        tap(1, 0, ys_ref, i, write_to=ya_ref)

    rdb, wrb = ya_ref, yb_ref
    for t in range(1, taps):
        for i in range(nblk):
            tap(1, t, rdb, i, write_to=wrb if t < taps - 1 else None)
        rdb, wrb = wrb, rdb


def kernel(x, lower_lp, upper_lp, W_up, W_low, W_h):
    B, N, C_in = x.shape
    T, C_out, _ = W_up.shape
    BC = B * C_in
    BCO = B * C_out
    nblk = N // _BR

    xt = jnp.transpose(x, (0, 2, 1)).reshape(BC, N)

    eye = jnp.eye(B, dtype=jnp.float32)
    wt = jnp.stack([
        jnp.stack([jnp.kron(eye, W_up[t]) for t in range(T)]),
        jnp.stack([jnp.kron(eye, W_low[t]) for t in range(T)]),
    ])
    wh = jnp.kron(eye, W_h)

    out = pl.pallas_call(
        functools.partial(_airtnn_body, nblk=nblk, taps=T),
        in_specs=[
            pl.BlockSpec(memory_space=pl.ANY),
            pl.BlockSpec(memory_space=pl.ANY),
            pl.BlockSpec((BC, N), lambda: (0, 0)),
            pl.BlockSpec((2, T, BCO, BC), lambda: (0, 0, 0, 0)),
            pl.BlockSpec((BCO, BC), lambda: (0, 0)),
        ],
        out_specs=pl.BlockSpec((BCO, N), lambda: (0, 0)),
        out_shape=jax.ShapeDtypeStruct((BCO, N), jnp.float32),
        scratch_shapes=[
            pltpu.VMEM((_NS, _BR, N), jnp.float32),
            pltpu.VMEM((nblk, _BR, N), jnp.bfloat16),
            pltpu.VMEM((BC, N), jnp.bfloat16),
            pltpu.VMEM((BC, N), jnp.bfloat16),
            pltpu.VMEM((BC, N), jnp.bfloat16),
            pltpu.SemaphoreType.DMA((2, nblk)),
        ],
        compiler_params=pltpu.CompilerParams(
            vmem_limit_bytes=60 * 1024 * 1024,
        ),
    )(upper_lp, lower_lp, xt, wt, wh)

    return jnp.transpose(out.reshape(B, C_out, N), (0, 2, 1))
